# baseline (speedup 1.0000x reference)
import dataclasses

import jax
import jax.numpy as jnp
from jax import lax
from jax.experimental import pallas as pl
from jax.experimental.pallas import tpu as pltpu
from jax.experimental.pallas import tpu_sc as plsc

_sc_params = pltpu.CompilerParams()
if "needs_layout_passes" in pltpu.CompilerParams.__dataclass_fields__:
    _sc_params = dataclasses.replace(_sc_params, needs_layout_passes=False)

N = 10000
E = 320000
R = 8
FEAT = 128
HID = 120
PAD = 128
NSEG = N * R
SEGROWS = 640
NC = 2
NS = 16
NW = NC * NS

F32 = jnp.float32
I32 = jnp.int32

_mesh = plsc.VectorSubcoreMesh(core_axis_name="c", subcore_axis_name="s")

CK = 400
SK = 400


def _prepare_body(src_hbm, dst_hbm, et_hbm, iota_hbm,
                  scale_hbm, gidx_hbm,
                  cnt_v, iota_v, dbuf, tbuf, sbuf, gout, sout, cnt_sh):
    c = lax.axis_index("c")
    s = lax.axis_index("s")
    wid = c * NS + s

    @pl.loop(0, SEGROWS)
    def _(i):
        for j in range(PAD // 16):
            cnt_v[i, pl.ds(j * 16, 16)] = jnp.zeros((16,), F32)

    rows_per = SEGROWS // NS
    pltpu.sync_copy(cnt_v.at[pl.ds(s * rows_per, rows_per)],
                    cnt_sh.at[pl.ds(s * rows_per, rows_per)])
    pltpu.sync_copy(iota_hbm, iota_v)
    plsc.subcore_barrier()

    ebase = s * (E // NS)

    @pl.loop(0, (E // NS) // CK)
    def _(b):
        base = ebase + b * CK
        pltpu.sync_copy(dst_hbm.at[pl.ds(base, CK)], dbuf)
        pltpu.sync_copy(et_hbm.at[pl.ds(base, CK)], tbuf)
        for g in range(CK // 16):
            dd = dbuf[pl.ds(g * 16, 16)]
            tt = tbuf[pl.ds(g * 16, 16)]
            seg = dd * R + tt
            row = lax.shift_right_logical(seg, 7)
            col = lax.bitwise_and(seg, 127)
            plsc.addupdate_scatter(cnt_v, [row, col], jnp.ones((16,), F32))

    for j in range(SEGROWS // PAD):
        pltpu.sync_copy(cnt_v.at[pl.ds(j * PAD, PAD)],
                        cnt_sh.at[iota_v.at[j]], add=True)
    plsc.subcore_barrier()

    pltpu.sync_copy(cnt_sh, cnt_v)

    @pl.loop(0, SEGROWS)
    def _(i):
        for j in range(PAD // 16):
            v = cnt_v[i, pl.ds(j * 16, 16)]
            cnt_v[i, pl.ds(j * 16, 16)] = 1.0 / jnp.maximum(v, 1.0)

    gbase = wid * (E // NW)

    @pl.loop(0, (E // NW) // SK)
    def _(b):
        base = gbase + b * SK
        pltpu.sync_copy(src_hbm.at[pl.ds(base, SK)], sbuf)
        pltpu.sync_copy(dst_hbm.at[pl.ds(base, SK)], dbuf)
        pltpu.sync_copy(et_hbm.at[pl.ds(base, SK)], tbuf)
        for g in range(SK // 16):
            sl = pl.ds(g * 16, 16)
            ss = sbuf[sl]
            dd = dbuf[sl]
            tt = tbuf[sl]
            gout[sl] = ss * R + tt
            si = dd * R + tt
            row = lax.shift_right_logical(si, 7)
            col = lax.bitwise_and(si, 127)
            sout[sl] = plsc.load_gather(cnt_v, [row, col])
        pltpu.sync_copy(gout, gidx_hbm.at[pl.ds(base, SK)])
        pltpu.sync_copy(sout, scale_hbm.at[pl.ds(base, SK)])


def _sc_prepare(src, dst, et, iota):
    return pl.kernel(
        _prepare_body,
        out_type=[jax.ShapeDtypeStruct((E,), F32),
                  jax.ShapeDtypeStruct((E,), I32)],
        mesh=_mesh,
        compiler_params=_sc_params,
        scratch_types=[
            pltpu.VMEM((SEGROWS, PAD), F32),
            pltpu.VMEM((SEGROWS // PAD, PAD), I32),
            pltpu.VMEM((CK,), I32),
            pltpu.VMEM((CK,), I32),
            pltpu.VMEM((SK,), I32),
            pltpu.VMEM((SK,), I32),
            pltpu.VMEM((SK,), F32),
            pltpu.VMEM_SHARED((SEGROWS, PAD), F32),
        ],
    )(src, dst, et, iota)


AK = 80
NB = E // NW // AK
NPAD = 10240
TRASH = NPAD
ACC_ROWS = NPAD + AK
ZR = 32
NG = AK // 16


def _agg_body(xw_hbm, gidx_hbm, dst_hbm, scale_hbm, part_hbm,
              gi0, gi1, gi2, di0, di1, di2, sc0, sc1, sc2,
              ridx0, ridx1, ridx2, ridxw, abuf, probe,
              rows0, rows1, rows2, zbuf, acc_sh,
              sm0, sm1, sm2, sg0, sg1, sg2, ss0, ss1, ss2, sw):
    c = lax.axis_index("c")
    s = lax.axis_index("s")
    wid = c * NS + s

    @pl.loop(0, ZR)
    def _(i):
        for j in range(PAD // 16):
            zbuf[i, pl.ds(j * 16, 16)] = jnp.zeros((16,), F32)

    rows_per = NPAD // NS
    for k in range(rows_per // ZR):
        pltpu.sync_copy(zbuf, acc_sh.at[pl.ds(s * rows_per + k * ZR, ZR)])
    plsc.subcore_barrier()

    ebase = wid * (E // NW)
    lane = lax.iota(I32, 16)
    gis, dis, scs = [gi0, gi1, gi2], [di0, di1, di2], [sc0, sc1, sc2]
    ridxs = [ridx0, ridx1, ridx2]
    rows, sems_m = [rows0, rows1, rows2], [sm0, sm1, sm2]
    sems_g, sems_s = [sg0, sg1, sg2], [ss0, ss1, ss2]

    def meta_copies(i, p):
        base = ebase + i * AK
        return [
            pltpu.make_async_copy(gidx_hbm.at[pl.ds(base, AK)],
                                  gis[p], sems_m[p]),
            pltpu.make_async_copy(scale_hbm.at[pl.ds(base, AK)],
                                  scs[p].at[pl.ds(16, AK)], sems_m[p]),
            pltpu.make_async_copy(dst_hbm.at[pl.ds(base, AK)],
                                  dis[p], sems_m[p]),
        ]

    def meta_issue(i, p):
        for d in meta_copies(i, p):
            d.start()

    def meta_wait(i, p):
        for d in meta_copies(i, p):
            d.wait()

    def gather_copy(p):
        return pltpu.make_async_copy(xw_hbm.at[gis[p]], rows[p], sems_g[p])

    def scatter_start(p):
        pltpu.async_copy(rows[p], acc_sh.at[ridxs[p]], sems_s[p], add=True)

    def scatter_wait(p):
        pltpu.make_async_copy(rows[p], acc_sh.at[ridxs[p]], sems_s[p]).wait()

    def compute(b, p):
        rv, dv, sv, rix = rows[p], dis[p], scs[p], ridxs[p]
        for e in range(AK):
            bc = plsc.load_gather(sv, [jnp.full((16,), 16 + e, I32)])
            for j in range(PAD // 16):
                sl = pl.ds(j * 16, 16)
                rv[e, sl] = rv[e, sl] * bc

        for g in range(NG):
            dd = dv[pl.ds(g * 16, 16)]
            plsc.store_scatter(probe, [dd], b * AK + g * 16 + lane)
        anyv = jnp.zeros((16,), I32)
        for g in range(NG):
            sl = pl.ds(g * 16, 16)
            dd = dv[sl]
            ev = b * AK + g * 16 + lane
            isrep = plsc.load_gather(probe, [dd]) == ev
            rix[sl] = jnp.where(isrep, dd, TRASH + g * 16 + lane)
            act = jnp.where(isrep, 0, 1)
            abuf[sl] = act
            anyv = anyv | act
        scatter_start(p)

        def _more(carry):
            for g in range(NG):
                sl = pl.ds(g * 16, 16)
                dd = dv[sl]
                m = abuf[sl] > 0
                plsc.store_scatter(probe, [dd], b * AK + g * 16 + lane,
                                   mask=m)
            nxt = jnp.zeros((16,), I32)
            for g in range(NG):
                sl = pl.ds(g * 16, 16)
                dd = dv[sl]
                ev = b * AK + g * 16 + lane
                m = abuf[sl] > 0
                isrep = m & (plsc.load_gather(probe, [dd]) == ev)
                ridxw[sl] = jnp.where(isrep, dd, TRASH + g * 16 + lane)
                act = jnp.where(isrep, 0, abuf[sl])
                abuf[sl] = act
                nxt = nxt | act
            pltpu.async_copy(rv, acc_sh.at[ridxw], sw, add=True).wait()
            return jnp.max(nxt)

        lax.while_loop(lambda nrem: nrem > 0, _more, jnp.max(anyv))

    meta_issue(0, 0)
    meta_issue(1, 1)
    meta_wait(0, 0)
    gather_copy(0).start()

    @pl.loop(0, NB // 3)
    def _(h):
        for q in range(3):
            i = h * 3 + q
            p = q
            pn = (q + 1) % 3
            meta_wait(i + 1, pn)

            @pl.when(i >= 2)
            def _():
                scatter_wait(pn)

            gather_copy(pn).start()
            gather_copy(p).wait()
            compute(i, p)

            @pl.when(i + 2 < NB)
            def _():
                meta_issue(i + 2, (q + 2) % 3)

    i0 = NB - 2
    meta_wait(i0 + 1, 1)
    scatter_wait(1)
    gather_copy(1).start()
    gather_copy(0).wait()
    compute(i0, 0)
    scatter_wait(2)
    gather_copy(1).wait()
    compute(i0 + 1, 1)
    scatter_wait(0)
    scatter_wait(1)

    plsc.subcore_barrier()
    for k in range(rows_per // ZR):
        r0 = s * rows_per + k * ZR
        pltpu.sync_copy(acc_sh.at[pl.ds(r0, ZR)],
                        part_hbm.at[pl.ds(c * NPAD + r0, ZR)])


def _sc_aggregate(xw, gidx, dst, scale):
    return pl.kernel(
        _agg_body,
        out_type=jax.ShapeDtypeStruct((NC * NPAD, PAD), F32),
        mesh=_mesh,
        compiler_params=_sc_params,
        scratch_types=(
            [pltpu.VMEM((AK,), I32)] * 3 +
            [pltpu.VMEM((AK,), I32)] * 3 +
            [pltpu.VMEM((16 + AK,), F32)] * 3 +
            [pltpu.VMEM((AK,), I32)] * 4 +
            [pltpu.VMEM((AK,), I32)] +
            [pltpu.VMEM((NPAD,), I32)] +
            [pltpu.VMEM((AK, PAD), F32)] * 3 +
            [pltpu.VMEM((ZR, PAD), F32)] +
            [pltpu.VMEM_SHARED((ACC_ROWS, PAD), F32)] +
            [pltpu.SemaphoreType.DMA] * 10
        ),
    )(xw, gidx, dst, scale)


BR = 400


def _pre1_body(x_ref, w_ref, root_ref, b_ref, xw_ref, xr_ref):
    xb = x_ref[...]
    for r in range(R):
        xw_ref[:, r, :] = jnp.dot(xb, w_ref[r], preferred_element_type=F32)
    xr_ref[...] = jnp.dot(xb, root_ref[...], preferred_element_type=F32) + b_ref[...]


def _tc_pre1(x, w1p, root1p, b1p):
    return pl.pallas_call(
        _pre1_body,
        grid=(N // BR,),
        in_specs=[
            pl.BlockSpec((BR, FEAT), lambda i: (i, 0)),
            pl.BlockSpec((R, FEAT, PAD), lambda i: (0, 0, 0)),
            pl.BlockSpec((FEAT, PAD), lambda i: (0, 0)),
            pl.BlockSpec((1, PAD), lambda i: (0, 0)),
        ],
        out_specs=[
            pl.BlockSpec((BR, R, PAD), lambda i: (i, 0, 0)),
            pl.BlockSpec((BR, PAD), lambda i: (i, 0)),
        ],
        out_shape=[jax.ShapeDtypeStruct((N, R, PAD), F32),
                   jax.ShapeDtypeStruct((N, PAD), F32)],
    )(x, w1p, root1p, b1p)


def _mid_body(p_ref, xr_ref, w_ref, root_ref, b_ref, xw_ref, xr2_ref):
    hb = jax.nn.relu(p_ref[0] + p_ref[1] + xr_ref[...])
    for r in range(R):
        xw_ref[:, r, :] = jnp.dot(hb, w_ref[r], preferred_element_type=F32)
    xr2_ref[...] = jnp.dot(hb, root_ref[...], preferred_element_type=F32) + b_ref[...]


def _tc_mid(part1, xr1, w2p, root2p, b2p):
    return pl.pallas_call(
        _mid_body,
        grid=(N // BR,),
        in_specs=[
            pl.BlockSpec((NC, BR, PAD), lambda i: (0, i, 0)),
            pl.BlockSpec((BR, PAD), lambda i: (i, 0)),
            pl.BlockSpec((R, PAD, PAD), lambda i: (0, 0, 0)),
            pl.BlockSpec((PAD, PAD), lambda i: (0, 0)),
            pl.BlockSpec((1, PAD), lambda i: (0, 0)),
        ],
        out_specs=[
            pl.BlockSpec((BR, R, PAD), lambda i: (i, 0, 0)),
            pl.BlockSpec((BR, PAD), lambda i: (i, 0)),
        ],
        out_shape=[jax.ShapeDtypeStruct((N, R, PAD), F32),
                   jax.ShapeDtypeStruct((N, PAD), F32)],
    )(part1, xr1, w2p, root2p, b2p)


def _final_body(p_ref, xr_ref, o_ref):
    o_ref[...] = (p_ref[0] + p_ref[1] + xr_ref[...])[:, :HID]


def _tc_final(part2, xr2):
    return pl.pallas_call(
        _final_body,
        grid=(N // BR,),
        in_specs=[
            pl.BlockSpec((NC, BR, PAD), lambda i: (0, i, 0)),
            pl.BlockSpec((BR, PAD), lambda i: (i, 0)),
        ],
        out_specs=pl.BlockSpec((BR, HID), lambda i: (i, 0)),
        out_shape=jax.ShapeDtypeStruct((N, HID), F32),
    )(part2, xr2)


def kernel(x, edge_index, edge_type, w1, root1, b1, w2, root2, b2):
    src = edge_index[0]
    dst = edge_index[1]
    et = edge_type.astype(I32)

    w1p = jnp.zeros((R, FEAT, PAD), F32)
    for i in range(8):
        w1p = w1p.at[:, i * 16:(i + 1) * 16, i * 15:(i + 1) * 15].set(w1[:, i])
    w2p = jnp.zeros((R, PAD, PAD), F32)
    for i in range(5):
        w2p = w2p.at[:, i * 24:(i + 1) * 24, i * 24:(i + 1) * 24].set(w2[:, i])
    root1p = jnp.pad(root1, ((0, 0), (0, PAD - HID)))
    b1p = jnp.pad(b1, (0, PAD - HID)).reshape(1, PAD)
    root2p = jnp.pad(root2, ((0, PAD - HID), (0, PAD - HID)))
    b2p = jnp.pad(b2, (0, PAD - HID)).reshape(1, PAD)

    iota = jnp.arange(SEGROWS, dtype=I32).reshape(SEGROWS // PAD, PAD)

    scale, gidx = _sc_prepare(src, dst, et, iota)
    xw1, xr1 = _tc_pre1(x, w1p, root1p, b1p)
    part1 = _sc_aggregate(xw1.reshape(NSEG, PAD), gidx, dst, scale)
    xw2, xr2 = _tc_mid(part1.reshape(NC, NPAD, PAD)[:, :N], xr1,
                       w2p, root2p, b2p)
    part2 = _sc_aggregate(xw2.reshape(NSEG, PAD), gidx, dst, scale)
    return _tc_final(part2.reshape(NC, NPAD, PAD)[:, :N], xr2)


# speedup vs baseline: 1.0446x; 1.0446x over previous
import dataclasses

import jax
import jax.numpy as jnp
from jax import lax
from jax.experimental import pallas as pl
from jax.experimental.pallas import tpu as pltpu
from jax.experimental.pallas import tpu_sc as plsc

_sc_params = pltpu.CompilerParams()
if "needs_layout_passes" in pltpu.CompilerParams.__dataclass_fields__:
    _sc_params = dataclasses.replace(_sc_params, needs_layout_passes=False)

N = 10000
E = 320000
R = 8
FEAT = 128
HID = 120
PAD = 128
NSEG = N * R
SEGROWS = 640
NC = 2
NS = 16
NW = NC * NS

F32 = jnp.float32
I32 = jnp.int32

_mesh = plsc.VectorSubcoreMesh(core_axis_name="c", subcore_axis_name="s")

CK = 400
SK = 400


def _prepare_body(src_hbm, dst_hbm, et_hbm, iota_hbm,
                  scale_hbm, gidx_hbm,
                  cnt_v, iota_v, dbuf, tbuf, sbuf, gout, sout, cnt_sh):
    c = lax.axis_index("c")
    s = lax.axis_index("s")
    wid = c * NS + s

    @pl.loop(0, SEGROWS)
    def _(i):
        for j in range(PAD // 16):
            cnt_v[i, pl.ds(j * 16, 16)] = jnp.zeros((16,), F32)

    rows_per = SEGROWS // NS
    pltpu.sync_copy(cnt_v.at[pl.ds(s * rows_per, rows_per)],
                    cnt_sh.at[pl.ds(s * rows_per, rows_per)])
    pltpu.sync_copy(iota_hbm, iota_v)
    plsc.subcore_barrier()

    ebase = s * (E // NS)

    @pl.loop(0, (E // NS) // CK)
    def _(b):
        base = ebase + b * CK
        pltpu.sync_copy(dst_hbm.at[pl.ds(base, CK)], dbuf)
        pltpu.sync_copy(et_hbm.at[pl.ds(base, CK)], tbuf)
        for g in range(CK // 16):
            dd = dbuf[pl.ds(g * 16, 16)]
            tt = tbuf[pl.ds(g * 16, 16)]
            seg = dd * R + tt
            row = lax.shift_right_logical(seg, 7)
            col = lax.bitwise_and(seg, 127)
            plsc.addupdate_scatter(cnt_v, [row, col], jnp.ones((16,), F32))

    for j in range(SEGROWS // PAD):
        pltpu.sync_copy(cnt_v.at[pl.ds(j * PAD, PAD)],
                        cnt_sh.at[iota_v.at[j]], add=True)
    plsc.subcore_barrier()

    pltpu.sync_copy(cnt_sh, cnt_v)

    @pl.loop(0, SEGROWS)
    def _(i):
        for j in range(PAD // 16):
            v = cnt_v[i, pl.ds(j * 16, 16)]
            cnt_v[i, pl.ds(j * 16, 16)] = 1.0 / jnp.maximum(v, 1.0)

    gbase = wid * (E // NW)

    @pl.loop(0, (E // NW) // SK)
    def _(b):
        base = gbase + b * SK
        pltpu.sync_copy(src_hbm.at[pl.ds(base, SK)], sbuf)
        pltpu.sync_copy(dst_hbm.at[pl.ds(base, SK)], dbuf)
        pltpu.sync_copy(et_hbm.at[pl.ds(base, SK)], tbuf)
        for g in range(SK // 16):
            sl = pl.ds(g * 16, 16)
            ss = sbuf[sl]
            dd = dbuf[sl]
            tt = tbuf[sl]
            gout[sl] = ss * R + tt
            si = dd * R + tt
            row = lax.shift_right_logical(si, 7)
            col = lax.bitwise_and(si, 127)
            sout[sl] = plsc.load_gather(cnt_v, [row, col])
        pltpu.sync_copy(gout, gidx_hbm.at[pl.ds(base, SK)])
        pltpu.sync_copy(sout, scale_hbm.at[pl.ds(base, SK)])


def _sc_prepare(src, dst, et, iota):
    return pl.kernel(
        _prepare_body,
        out_type=[jax.ShapeDtypeStruct((E,), F32),
                  jax.ShapeDtypeStruct((E,), I32)],
        mesh=_mesh,
        compiler_params=_sc_params,
        scratch_types=[
            pltpu.VMEM((SEGROWS, PAD), F32),
            pltpu.VMEM((SEGROWS // PAD, PAD), I32),
            pltpu.VMEM((CK,), I32),
            pltpu.VMEM((CK,), I32),
            pltpu.VMEM((SK,), I32),
            pltpu.VMEM((SK,), I32),
            pltpu.VMEM((SK,), F32),
            pltpu.VMEM_SHARED((SEGROWS, PAD), F32),
        ],
    )(src, dst, et, iota)


AK = 80
NB = E // NW // AK
NPAD = 10240
TRASH = NPAD
ACC_ROWS = NPAD + AK
ZR = 128
NG = AK // 16


def _agg_body(xw_hbm, gidx_hbm, dst_hbm, scale_hbm, part_hbm,
              gi0, gi1, di0, di1, sc0, sc1, ridx_v, abuf, probe,
              rows0, rows1, zbuf, acc_sh, sm0, sm1, sg0, sg1):
    c = lax.axis_index("c")
    s = lax.axis_index("s")
    wid = c * NS + s

    @pl.loop(0, ZR)
    def _(i):
        for j in range(PAD // 16):
            zbuf[i, pl.ds(j * 16, 16)] = jnp.zeros((16,), F32)

    rows_per = NPAD // NS
    for k in range(rows_per // ZR):
        pltpu.sync_copy(zbuf, acc_sh.at[pl.ds(s * rows_per + k * ZR, ZR)])
    plsc.subcore_barrier()

    ebase = wid * (E // NW)
    lane = lax.iota(I32, 16)
    gis, dis, scs = [gi0, gi1], [di0, di1], [sc0, sc1]
    rows, sems_m, sems_g = [rows0, rows1], [sm0, sm1], [sg0, sg1]

    def meta_copies(i, p):
        base = ebase + i * AK
        return [
            pltpu.make_async_copy(gidx_hbm.at[pl.ds(base, AK)],
                                  gis[p], sems_m[p]),
            pltpu.make_async_copy(scale_hbm.at[pl.ds(base, AK)],
                                  scs[p].at[pl.ds(16, AK)], sems_m[p]),
            pltpu.make_async_copy(dst_hbm.at[pl.ds(base, AK)],
                                  dis[p], sems_m[p]),
        ]

    def meta_issue(i, p):
        for d in meta_copies(i, p):
            d.start()

    def meta_wait(i, p):
        for d in meta_copies(i, p):
            d.wait()

    def gather_copy(p):
        return pltpu.make_async_copy(xw_hbm.at[gis[p]], rows[p], sems_g[p])

    def compute(b, p):
        rv, dv, sv = rows[p], dis[p], scs[p]
        for e in range(AK):
            bc = plsc.load_gather(sv, [jnp.full((16,), 16 + e, I32)])
            for j in range(PAD // 16):
                sl = pl.ds(j * 16, 16)
                rv[e, sl] = rv[e, sl] * bc

        for g in range(NG):
            dd = dv[pl.ds(g * 16, 16)]
            plsc.store_scatter(probe, [dd], b * AK + g * 16 + lane)
        anyv = jnp.zeros((16,), I32)
        for g in range(NG):
            sl = pl.ds(g * 16, 16)
            dd = dv[sl]
            ev = b * AK + g * 16 + lane
            isrep = plsc.load_gather(probe, [dd]) == ev
            ridx_v[sl] = jnp.where(isrep, dd, TRASH + g * 16 + lane)
            act = jnp.where(isrep, 0, 1)
            abuf[sl] = act
            anyv = anyv | act
        pltpu.sync_copy(rv, acc_sh.at[ridx_v], add=True)

        def _more(carry):
            for g in range(NG):
                sl = pl.ds(g * 16, 16)
                dd = dv[sl]
                m = abuf[sl] > 0
                plsc.store_scatter(probe, [dd], b * AK + g * 16 + lane,
                                   mask=m)
            nxt = jnp.zeros((16,), I32)
            for g in range(NG):
                sl = pl.ds(g * 16, 16)
                dd = dv[sl]
                ev = b * AK + g * 16 + lane
                m = abuf[sl] > 0
                isrep = m & (plsc.load_gather(probe, [dd]) == ev)
                ridx_v[sl] = jnp.where(isrep, dd, TRASH + g * 16 + lane)
                act = jnp.where(isrep, 0, abuf[sl])
                abuf[sl] = act
                nxt = nxt | act
            pltpu.sync_copy(rv, acc_sh.at[ridx_v], add=True)
            return jnp.max(nxt)

        lax.while_loop(lambda nrem: nrem > 0, _more, jnp.max(anyv))

    meta_issue(0, 0)
    meta_issue(1, 1)
    meta_wait(0, 0)
    gather_copy(0).start()

    @pl.loop(0, NB // 2)
    def _(h):
        for p in range(2):
            i = h * 2 + p
            pn = 1 - p
            meta_wait(i + 1, pn)
            gather_copy(pn).start()
            gather_copy(p).wait()
            compute(i, p)

            @pl.when(i + 2 < NB)
            def _():
                meta_issue(i + 2, p)

    gather_copy((NB - 1) % 2).wait()
    compute(NB - 1, (NB - 1) % 2)

    plsc.subcore_barrier()
    for k in range(rows_per // ZR):
        r0 = s * rows_per + k * ZR
        pltpu.sync_copy(acc_sh.at[pl.ds(r0, ZR)],
                        part_hbm.at[pl.ds(c * NPAD + r0, ZR)])


def _sc_aggregate(xw, gidx, dst, scale):
    return pl.kernel(
        _agg_body,
        out_type=jax.ShapeDtypeStruct((NC * NPAD, PAD), F32),
        mesh=_mesh,
        compiler_params=_sc_params,
        scratch_types=[
            pltpu.VMEM((AK,), I32),
            pltpu.VMEM((AK,), I32),
            pltpu.VMEM((AK,), I32),
            pltpu.VMEM((AK,), I32),
            pltpu.VMEM((16 + AK,), F32),
            pltpu.VMEM((16 + AK,), F32),
            pltpu.VMEM((AK,), I32),
            pltpu.VMEM((AK,), I32),
            pltpu.VMEM((NPAD,), I32),
            pltpu.VMEM((AK, PAD), F32),
            pltpu.VMEM((AK, PAD), F32),
            pltpu.VMEM((ZR, PAD), F32),
            pltpu.VMEM_SHARED((ACC_ROWS, PAD), F32),
            pltpu.SemaphoreType.DMA,
            pltpu.SemaphoreType.DMA,
            pltpu.SemaphoreType.DMA,
            pltpu.SemaphoreType.DMA,
        ],
    )(xw, gidx, dst, scale)


BR = 400


def _pre1_body(x_ref, w_ref, root_ref, b_ref, xw_ref, xr_ref):
    xb = x_ref[...]
    for r in range(R):
        xw_ref[:, r, :] = jnp.dot(xb, w_ref[r], preferred_element_type=F32)
    xr_ref[...] = jnp.dot(xb, root_ref[...], preferred_element_type=F32) + b_ref[...]


def _tc_pre1(x, w1p, root1p, b1p):
    return pl.pallas_call(
        _pre1_body,
        grid=(N // BR,),
        in_specs=[
            pl.BlockSpec((BR, FEAT), lambda i: (i, 0)),
            pl.BlockSpec((R, FEAT, PAD), lambda i: (0, 0, 0)),
            pl.BlockSpec((FEAT, PAD), lambda i: (0, 0)),
            pl.BlockSpec((1, PAD), lambda i: (0, 0)),
        ],
        out_specs=[
            pl.BlockSpec((BR, R, PAD), lambda i: (i, 0, 0)),
            pl.BlockSpec((BR, PAD), lambda i: (i, 0)),
        ],
        out_shape=[jax.ShapeDtypeStruct((N, R, PAD), F32),
                   jax.ShapeDtypeStruct((N, PAD), F32)],
    )(x, w1p, root1p, b1p)


def _mid_body(p_ref, xr_ref, w_ref, root_ref, b_ref, xw_ref, xr2_ref):
    hb = jax.nn.relu(p_ref[0] + p_ref[1] + xr_ref[...])
    for r in range(R):
        xw_ref[:, r, :] = jnp.dot(hb, w_ref[r], preferred_element_type=F32)
    xr2_ref[...] = jnp.dot(hb, root_ref[...], preferred_element_type=F32) + b_ref[...]


def _tc_mid(part1, xr1, w2p, root2p, b2p):
    return pl.pallas_call(
        _mid_body,
        grid=(N // BR,),
        in_specs=[
            pl.BlockSpec((NC, BR, PAD), lambda i: (0, i, 0)),
            pl.BlockSpec((BR, PAD), lambda i: (i, 0)),
            pl.BlockSpec((R, PAD, PAD), lambda i: (0, 0, 0)),
            pl.BlockSpec((PAD, PAD), lambda i: (0, 0)),
            pl.BlockSpec((1, PAD), lambda i: (0, 0)),
        ],
        out_specs=[
            pl.BlockSpec((BR, R, PAD), lambda i: (i, 0, 0)),
            pl.BlockSpec((BR, PAD), lambda i: (i, 0)),
        ],
        out_shape=[jax.ShapeDtypeStruct((N, R, PAD), F32),
                   jax.ShapeDtypeStruct((N, PAD), F32)],
    )(part1, xr1, w2p, root2p, b2p)


def _final_body(p_ref, xr_ref, o_ref):
    o_ref[...] = (p_ref[0] + p_ref[1] + xr_ref[...])[:, :HID]


def _tc_final(part2, xr2):
    return pl.pallas_call(
        _final_body,
        grid=(N // BR,),
        in_specs=[
            pl.BlockSpec((NC, BR, PAD), lambda i: (0, i, 0)),
            pl.BlockSpec((BR, PAD), lambda i: (i, 0)),
        ],
        out_specs=pl.BlockSpec((BR, HID), lambda i: (i, 0)),
        out_shape=jax.ShapeDtypeStruct((N, HID), F32),
    )(part2, xr2)


def kernel(x, edge_index, edge_type, w1, root1, b1, w2, root2, b2):
    src = edge_index[0]
    dst = edge_index[1]
    et = edge_type.astype(I32)

    w1p = jnp.zeros((R, FEAT, PAD), F32)
    for i in range(8):
        w1p = w1p.at[:, i * 16:(i + 1) * 16, i * 15:(i + 1) * 15].set(w1[:, i])
    w2p = jnp.zeros((R, PAD, PAD), F32)
    for i in range(5):
        w2p = w2p.at[:, i * 24:(i + 1) * 24, i * 24:(i + 1) * 24].set(w2[:, i])
    root1p = jnp.pad(root1, ((0, 0), (0, PAD - HID)))
    b1p = jnp.pad(b1, (0, PAD - HID)).reshape(1, PAD)
    root2p = jnp.pad(root2, ((0, PAD - HID), (0, PAD - HID)))
    b2p = jnp.pad(b2, (0, PAD - HID)).reshape(1, PAD)

    iota = jnp.arange(SEGROWS, dtype=I32).reshape(SEGROWS // PAD, PAD)

    scale, gidx = _sc_prepare(src, dst, et, iota)
    xw1, xr1 = _tc_pre1(x, w1p, root1p, b1p)
    part1 = _sc_aggregate(xw1.reshape(NSEG, PAD), gidx, dst, scale)
    xw2, xr2 = _tc_mid(part1.reshape(NC, NPAD, PAD)[:, :N], xr1,
                       w2p, root2p, b2p)
    part2 = _sc_aggregate(xw2.reshape(NSEG, PAD), gidx, dst, scale)
    return _tc_final(part2.reshape(NC, NPAD, PAD)[:, :N], xr2)
